# baseline (speedup 1.0000x reference)
import jax
import jax.numpy as jnp
from jax.experimental import pallas as pl

_SAMPLE_N = 16384
_FEAT = 128
_BLOCK = 16384


def _copy_body(x_ref, o_ref):
    o_ref[...] = x_ref[...]


def kernel(dataset):
    return pl.pallas_call(
        _copy_body,
        grid=(_SAMPLE_N // _BLOCK,),
        in_specs=[pl.BlockSpec((_BLOCK, _FEAT), lambda i: (i, 0))],
        out_specs=pl.BlockSpec((_BLOCK, _FEAT), lambda i: (i, 0)),
        out_shape=jax.ShapeDtypeStruct((_SAMPLE_N, _FEAT), jnp.float32),
    )(dataset)
